# baseline (speedup 1.0000x reference)
import jax
import jax.numpy as jnp
from jax import lax
from jax.experimental import pallas as pl
from jax.experimental.pallas import tpu as pltpu
from jax.experimental.pallas import tpu_sc as plsc

_LOG_REG_C = 2.5e-05
_LEAF_COEF = 10.0

_NTRANS = 50000
_N_PARENTS = 800000
_N_STEMS = 400000
_NCOLS = 105

_NW = 32
_CHUNK_A = _N_PARENTS // _NW
_CHUNK_A_BUF = 25008

_C_BLK = 8192
_BLKS_1A = 38
_N1 = _BLKS_1A * _C_BLK
_N2 = _N_STEMS - _N1
_GRID1B = (_N2 + _C_BLK - 1) // _C_BLK

_CHUNK_B1 = _N1 // _NW
_CHUNK_B2 = 2784
_CHUNK_B2_TAIL = _N2 - 31 * _CHUNK_B2


def _tc1_body(stem_ref, v_ref):
    v_ref[...] = jnp.sum(jnp.exp(stem_ref[...]), axis=0)


def _make_tc1(grid, blk_off, out_n):
    return pl.pallas_call(
        _tc1_body,
        grid=(grid,),
        in_specs=[pl.BlockSpec((_NCOLS, _C_BLK), lambda i: (0, i + blk_off))],
        out_specs=pl.BlockSpec((_C_BLK,), lambda i: (i,)),
        out_shape=jax.ShapeDtypeStruct((out_n,), jnp.float32),
    )


def _sca_body(qsa_hbm, pb_hbm, zeros_hbm, out_hbm, vals, idx, acc):
    c = lax.axis_index("c")
    s = lax.axis_index("s")
    wid = c * 16 + s

    @pl.when(s == 0)
    def _():
        pltpu.sync_copy(zeros_hbm, acc)

    pltpu.sync_copy(qsa_hbm.at[pl.ds(wid * _CHUNK_A, _CHUNK_A)],
                    vals.at[pl.ds(0, _CHUNK_A)])
    pltpu.sync_copy(pb_hbm.at[pl.ds(wid * _CHUNK_A, _CHUNK_A)],
                    idx.at[pl.ds(0, _CHUNK_A)])

    def _exp_step(i, carry):
        sl = pl.ds(i * 16, 16)
        vals[sl] = jnp.exp(vals[sl])
        return carry

    lax.fori_loop(0, _CHUNK_A_BUF // 16, _exp_step, 0)

    lane = lax.iota(jnp.int32, 16)
    tail = pl.ds(_CHUNK_A_BUF - 16, 16)
    vals[tail] = jnp.where(lane < 8, vals[tail], 0.0)
    idx[tail] = jnp.where(lane < 8, idx[tail], 0)

    plsc.subcore_barrier()
    pltpu.sync_copy(vals, acc.at[idx], add=True)
    plsc.subcore_barrier()

    @pl.when(s == 0)
    def _():
        pltpu.sync_copy(acc, out_hbm.at[c])


_sc_a = pl.kernel(
    _sca_body,
    mesh=plsc.VectorSubcoreMesh(core_axis_name="c", subcore_axis_name="s"),
    out_type=jax.ShapeDtypeStruct((2, _NTRANS), jnp.float32),
    scratch_types=[
        pltpu.VMEM((_CHUNK_A_BUF,), jnp.float32),
        pltpu.VMEM((_CHUNK_A_BUF,), jnp.int32),
        pltpu.VMEM_SHARED((_NTRANS,), jnp.float32),
    ],
)


def _make_scb(chunk, tail, sb_off):
    def _scb_body(v_hbm, sb_hbm, zeros_hbm, out_hbm, vals, idx, acc):
        c = lax.axis_index("c")
        s = lax.axis_index("s")
        wid = c * 16 + s

        @pl.when(s == 0)
        def _():
            pltpu.sync_copy(zeros_hbm, acc)

        if tail == chunk:
            pltpu.sync_copy(v_hbm.at[pl.ds(wid * chunk, chunk)], vals)
            pltpu.sync_copy(sb_hbm.at[pl.ds(sb_off + wid * chunk, chunk)], idx)
        else:
            @pl.when(wid < 31)
            def _():
                pltpu.sync_copy(v_hbm.at[pl.ds(wid * chunk, chunk)], vals)
                pltpu.sync_copy(sb_hbm.at[pl.ds(sb_off + wid * chunk, chunk)],
                                idx)

            @pl.when(wid == 31)
            def _():
                pltpu.sync_copy(v_hbm.at[pl.ds(31 * chunk, tail)],
                                vals.at[pl.ds(0, tail)])
                pltpu.sync_copy(sb_hbm.at[pl.ds(sb_off + 31 * chunk, tail)],
                                idx.at[pl.ds(0, tail)])
                zf = jnp.zeros((16,), jnp.float32)
                zi = jnp.zeros((16,), jnp.int32)

                def _fill(i, carry):
                    sl = pl.ds(i * 16, 16)
                    vals[sl] = zf
                    idx[sl] = zi
                    return carry

                lax.fori_loop(tail // 16, chunk // 16, _fill, 0)

        plsc.subcore_barrier()
        pltpu.sync_copy(vals, acc.at[idx], add=True)
        plsc.subcore_barrier()

        @pl.when(s == 0)
        def _():
            pltpu.sync_copy(acc, out_hbm.at[c])

    return pl.kernel(
        _scb_body,
        mesh=plsc.VectorSubcoreMesh(core_axis_name="c", subcore_axis_name="s"),
        out_type=jax.ShapeDtypeStruct((2, _NTRANS), jnp.float32),
        scratch_types=[
            pltpu.VMEM((chunk,), jnp.float32),
            pltpu.VMEM((chunk,), jnp.int32),
            pltpu.VMEM_SHARED((_NTRANS,), jnp.float32),
        ],
    )


_sc_b1 = _make_scb(_CHUNK_B1, _CHUNK_B1, 0)
_sc_b2 = _make_scb(_CHUNK_B2, _CHUNK_B2_TAIL, _N1)


def _tc2_body(pa_ref, pb1_ref, pb2_ref, mol_ref, r_ref, d_ref,
              loss_ref, term_ref, flow_ref):
    exp_inflow = jnp.sum(pa_ref[...], axis=0, keepdims=True)
    inflow = jnp.log(exp_inflow + _LOG_REG_C)
    exp_outflow = (jnp.sum(pb1_ref[...], axis=0, keepdims=True)
                   + jnp.sum(pb2_ref[...], axis=0, keepdims=True)
                   + jnp.exp(mol_ref[...]))
    dd = d_ref[...]
    opr = jnp.log(_LOG_REG_C + r_ref[...] + exp_outflow * (1.0 - dd))
    losses = (inflow - opr) ** 2
    term = jnp.sum(losses * dd) / (jnp.sum(dd) + 1e-20)
    flow = jnp.sum(losses * (1.0 - dd)) / (jnp.sum(1.0 - dd) + 1e-20)
    loss_ref[0, 0] = term * _LEAF_COEF + flow
    term_ref[0, 0] = term
    flow_ref[0, 0] = flow


def _tc2(part_a, part_b1, part_b2, mol2, r2, d2):
    return pl.pallas_call(
        _tc2_body,
        out_specs=[
            pl.BlockSpec(memory_space=pltpu.SMEM),
            pl.BlockSpec(memory_space=pltpu.SMEM),
            pl.BlockSpec(memory_space=pltpu.SMEM),
        ],
        out_shape=[
            jax.ShapeDtypeStruct((1, 1), jnp.float32),
            jax.ShapeDtypeStruct((1, 1), jnp.float32),
            jax.ShapeDtypeStruct((1, 1), jnp.float32),
        ],
    )(part_a, part_b1, part_b2, mol2, r2, d2)


def kernel(stem_out_s, mol_out_s, qsa_p, r, d, pb, stem_batch):
    zeros = jnp.zeros((_NTRANS,), jnp.float32)
    sb = stem_batch.astype(jnp.int32)
    part_a = _sc_a(qsa_p, pb.astype(jnp.int32), zeros)

    stem_t = stem_out_s.T
    v1 = _make_tc1(_BLKS_1A, 0, _N1)(stem_t)

    v1, part_a = lax.optimization_barrier((v1, part_a))
    part_b1 = _sc_b1(v1, sb, zeros)

    stem_t, v1 = lax.optimization_barrier((stem_t, v1))
    v2 = _make_tc1(_GRID1B, _BLKS_1A, _N2)(stem_t)
    v2, part_b1 = lax.optimization_barrier((v2, part_b1))
    part_b2 = _sc_b2(v2, sb, zeros)

    mol2 = mol_out_s.reshape(1, _NTRANS)
    r2 = r.reshape(1, _NTRANS)
    d2 = d.reshape(1, _NTRANS)
    loss, term, flow = _tc2(part_a, part_b1, part_b2, mol2, r2, d2)
    return (loss[0, 0], term[0, 0], flow[0, 0])


# speedup vs baseline: 1.1193x; 1.1193x over previous
import jax
import jax.numpy as jnp
from jax import lax
from jax.experimental import pallas as pl
from jax.experimental.pallas import tpu as pltpu
from jax.experimental.pallas import tpu_sc as plsc

_LOG_REG_C = 2.5e-05
_LEAF_COEF = 10.0

_NTRANS = 50000
_N_PARENTS = 800000
_N_STEMS = 400000
_NCOLS = 105

_NW = 32
_CHUNK_A = _N_PARENTS // _NW
_CHUNK_A_BUF = 25008

_C_BLK = 16384
_BLKS_1A = 19
_N1 = _BLKS_1A * _C_BLK
_N2 = _N_STEMS - _N1
_GRID1B = (_N2 + _C_BLK - 1) // _C_BLK

_CHUNK_B1 = _N1 // _NW
_CHUNK_B2 = 2784
_CHUNK_B2_TAIL = _N2 - 31 * _CHUNK_B2


def _tc1_body(stem_ref, v_ref):
    v_ref[...] = jnp.sum(jnp.exp(stem_ref[...]), axis=0)


def _make_tc1(grid, blk_off, out_n):
    return pl.pallas_call(
        _tc1_body,
        grid=(grid,),
        in_specs=[pl.BlockSpec((_NCOLS, _C_BLK), lambda i: (0, i + blk_off))],
        out_specs=pl.BlockSpec((_C_BLK,), lambda i: (i,)),
        out_shape=jax.ShapeDtypeStruct((out_n,), jnp.float32),
    )


def _sca_body(qsa_hbm, pb_hbm, zeros_hbm, out_hbm, vals, idx, acc):
    c = lax.axis_index("c")
    s = lax.axis_index("s")
    wid = c * 16 + s

    @pl.when(s == 0)
    def _():
        pltpu.sync_copy(zeros_hbm, acc)

    pltpu.sync_copy(qsa_hbm.at[pl.ds(wid * _CHUNK_A, _CHUNK_A)],
                    vals.at[pl.ds(0, _CHUNK_A)])
    pltpu.sync_copy(pb_hbm.at[pl.ds(wid * _CHUNK_A, _CHUNK_A)],
                    idx.at[pl.ds(0, _CHUNK_A)])

    def _exp_step(i, carry):
        sl = pl.ds(i * 16, 16)
        vals[sl] = jnp.exp(vals[sl])
        return carry

    lax.fori_loop(0, _CHUNK_A_BUF // 16, _exp_step, 0)

    lane = lax.iota(jnp.int32, 16)
    tail = pl.ds(_CHUNK_A_BUF - 16, 16)
    vals[tail] = jnp.where(lane < 8, vals[tail], 0.0)
    idx[tail] = jnp.where(lane < 8, idx[tail], 0)

    plsc.subcore_barrier()
    pltpu.sync_copy(vals, acc.at[idx], add=True)
    plsc.subcore_barrier()

    @pl.when(s == 0)
    def _():
        pltpu.sync_copy(acc, out_hbm.at[c])


_sc_a = pl.kernel(
    _sca_body,
    mesh=plsc.VectorSubcoreMesh(core_axis_name="c", subcore_axis_name="s"),
    out_type=jax.ShapeDtypeStruct((2, _NTRANS), jnp.float32),
    scratch_types=[
        pltpu.VMEM((_CHUNK_A_BUF,), jnp.float32),
        pltpu.VMEM((_CHUNK_A_BUF,), jnp.int32),
        pltpu.VMEM_SHARED((_NTRANS,), jnp.float32),
    ],
)


def _make_scb(chunk, tail, sb_off):
    def _scb_body(v_hbm, sb_hbm, zeros_hbm, out_hbm, vals, idx, acc):
        c = lax.axis_index("c")
        s = lax.axis_index("s")
        wid = c * 16 + s

        @pl.when(s == 0)
        def _():
            pltpu.sync_copy(zeros_hbm, acc)

        if tail == chunk:
            pltpu.sync_copy(v_hbm.at[pl.ds(wid * chunk, chunk)], vals)
            pltpu.sync_copy(sb_hbm.at[pl.ds(sb_off + wid * chunk, chunk)], idx)
        else:
            @pl.when(wid < 31)
            def _():
                pltpu.sync_copy(v_hbm.at[pl.ds(wid * chunk, chunk)], vals)
                pltpu.sync_copy(sb_hbm.at[pl.ds(sb_off + wid * chunk, chunk)],
                                idx)

            @pl.when(wid == 31)
            def _():
                pltpu.sync_copy(v_hbm.at[pl.ds(31 * chunk, tail)],
                                vals.at[pl.ds(0, tail)])
                pltpu.sync_copy(sb_hbm.at[pl.ds(sb_off + 31 * chunk, tail)],
                                idx.at[pl.ds(0, tail)])
                zf = jnp.zeros((16,), jnp.float32)
                zi = jnp.zeros((16,), jnp.int32)

                def _fill(i, carry):
                    sl = pl.ds(i * 16, 16)
                    vals[sl] = zf
                    idx[sl] = zi
                    return carry

                lax.fori_loop(tail // 16, chunk // 16, _fill, 0)

        plsc.subcore_barrier()
        pltpu.sync_copy(vals, acc.at[idx], add=True)
        plsc.subcore_barrier()

        @pl.when(s == 0)
        def _():
            pltpu.sync_copy(acc, out_hbm.at[c])

    return pl.kernel(
        _scb_body,
        mesh=plsc.VectorSubcoreMesh(core_axis_name="c", subcore_axis_name="s"),
        out_type=jax.ShapeDtypeStruct((2, _NTRANS), jnp.float32),
        scratch_types=[
            pltpu.VMEM((chunk,), jnp.float32),
            pltpu.VMEM((chunk,), jnp.int32),
            pltpu.VMEM_SHARED((_NTRANS,), jnp.float32),
        ],
    )


_sc_b1 = _make_scb(_CHUNK_B1, _CHUNK_B1, 0)
_sc_b2 = _make_scb(_CHUNK_B2, _CHUNK_B2_TAIL, _N1)


def _tc2_body(pa_ref, pb1_ref, pb2_ref, mol_ref, r_ref, d_ref,
              loss_ref, term_ref, flow_ref):
    exp_inflow = jnp.sum(pa_ref[...], axis=0, keepdims=True)
    inflow = jnp.log(exp_inflow + _LOG_REG_C)
    exp_outflow = (jnp.sum(pb1_ref[...], axis=0, keepdims=True)
                   + jnp.sum(pb2_ref[...], axis=0, keepdims=True)
                   + jnp.exp(mol_ref[...]))
    dd = d_ref[...]
    opr = jnp.log(_LOG_REG_C + r_ref[...] + exp_outflow * (1.0 - dd))
    losses = (inflow - opr) ** 2
    term = jnp.sum(losses * dd) / (jnp.sum(dd) + 1e-20)
    flow = jnp.sum(losses * (1.0 - dd)) / (jnp.sum(1.0 - dd) + 1e-20)
    loss_ref[0, 0] = term * _LEAF_COEF + flow
    term_ref[0, 0] = term
    flow_ref[0, 0] = flow


def _tc2(part_a, part_b1, part_b2, mol2, r2, d2):
    return pl.pallas_call(
        _tc2_body,
        out_specs=[
            pl.BlockSpec(memory_space=pltpu.SMEM),
            pl.BlockSpec(memory_space=pltpu.SMEM),
            pl.BlockSpec(memory_space=pltpu.SMEM),
        ],
        out_shape=[
            jax.ShapeDtypeStruct((1, 1), jnp.float32),
            jax.ShapeDtypeStruct((1, 1), jnp.float32),
            jax.ShapeDtypeStruct((1, 1), jnp.float32),
        ],
    )(part_a, part_b1, part_b2, mol2, r2, d2)


def kernel(stem_out_s, mol_out_s, qsa_p, r, d, pb, stem_batch):
    zeros = jnp.zeros((_NTRANS,), jnp.float32)
    sb = stem_batch.astype(jnp.int32)
    part_a = _sc_a(qsa_p, pb.astype(jnp.int32), zeros)

    stem_t = stem_out_s.T
    v1 = _make_tc1(_BLKS_1A, 0, _N1)(stem_t)

    v1, part_a = lax.optimization_barrier((v1, part_a))
    part_b1 = _sc_b1(v1, sb, zeros)

    stem_t, v1 = lax.optimization_barrier((stem_t, v1))
    v2 = _make_tc1(_GRID1B, _BLKS_1A, _N2)(stem_t)
    v2, part_b1 = lax.optimization_barrier((v2, part_b1))
    part_b2 = _sc_b2(v2, sb, zeros)

    mol2 = mol_out_s.reshape(1, _NTRANS)
    r2 = r.reshape(1, _NTRANS)
    d2 = d.reshape(1, _NTRANS)
    loss, term, flow = _tc2(part_a, part_b1, part_b2, mol2, r2, d2)
    return (loss[0, 0], term[0, 0], flow[0, 0])
